# Initial kernel scaffold; baseline (speedup 1.0000x reference)
import jax
import jax.numpy as jnp
import numpy as np
from jax import lax
from jax.experimental import pallas as pl
from jax.experimental.pallas import tpu as pltpu
from jax.experimental.pallas import tpu_sc as plsc

N = 10000
E = 320000
IN_DIM = 128
H = 8
D = 16
HD = H * D
LEAKY_SLOPE = 0.2

NC = 2
NS = 16
NW = NC * NS
EPW = E // NW
K = 80
NCHUNK = EPW // K
RPT = N // NS

_f32 = jnp.float32
_i32 = jnp.int32


def _mesh():
    return plsc.VectorSubcoreMesh(core_axis_name="c", subcore_axis_name="s")


def _mm_body(x_ref, wp_ref, wm_ref, wh_ref, ssrc_ref, sdst_ref):
    xb = x_ref[...]
    wh_ref[...] = jnp.dot(xb, wp_ref[...], preferred_element_type=_f32)
    s = jnp.dot(xb, wm_ref[...], preferred_element_type=_f32)
    ssrc_ref[...] = s[:, :16]
    sdst_ref[...] = s[:, 16:]


def _precompute(x, wp, wm, blk=400):
    grid = (N // blk,)
    return pl.pallas_call(
        _mm_body,
        grid=grid,
        in_specs=[
            pl.BlockSpec((blk, IN_DIM), lambda i: (i, 0)),
            pl.BlockSpec((IN_DIM, HD), lambda i: (0, 0)),
            pl.BlockSpec((IN_DIM, 32), lambda i: (0, 0)),
        ],
        out_specs=[
            pl.BlockSpec((blk, HD), lambda i: (i, 0)),
            pl.BlockSpec((blk, 16), lambda i: (i, 0)),
            pl.BlockSpec((blk, 16), lambda i: (i, 0)),
        ],
        out_shape=[
            jax.ShapeDtypeStruct((N, HD), _f32),
            jax.ShapeDtypeStruct((N, 16), _f32),
            jax.ShapeDtypeStruct((N, 16), _f32),
        ],
    )(x, wp, wm)


def _scores_body(src_hbm, dst_hbm, ssrc_hbm, sdst_hbm,
                 eexp_hbm, esum_hbm,
                 sidx, didx, sbuf, dbuf, ebuf, zbuf, esum_sh, sem1, sem2):
    c = lax.axis_index("c")
    s = lax.axis_index("s")

    def zrow(i, carry):
        zbuf[i, :] = jnp.zeros((16,), _f32)
        return carry
    lax.fori_loop(0, RPT, zrow, 0)
    pltpu.sync_copy(zbuf, esum_sh.at[pl.ds(s * RPT, RPT)])
    plsc.subcore_barrier()

    base0 = (c * NS + s) * EPW

    def chunk(ci, carry):
        base = base0 + ci * K
        pltpu.sync_copy(src_hbm.at[pl.ds(base, K)], sidx)
        pltpu.sync_copy(dst_hbm.at[pl.ds(base, K)], didx)
        h1 = pltpu.async_copy(ssrc_hbm.at[sidx], sbuf, sem1)
        h2 = pltpu.async_copy(sdst_hbm.at[didx], dbuf, sem2)
        h1.wait()
        h2.wait()

        def edge(k, carry2):
            e = sbuf[k, :] + dbuf[k, :]
            e = jnp.where(e >= 0.0, e, LEAKY_SLOPE * e)
            ebuf[k, :] = jnp.exp(e)
            return carry2
        lax.fori_loop(0, K, edge, 0)

        pltpu.sync_copy(ebuf, eexp_hbm.at[pl.ds(base, K)])
        pltpu.sync_copy(ebuf, esum_sh.at[didx], add=True)
        return carry
    lax.fori_loop(0, NCHUNK, chunk, 0)

    plsc.subcore_barrier()
    pltpu.sync_copy(esum_sh.at[pl.ds(s * RPT, RPT)],
                    esum_hbm.at[c, pl.ds(s * RPT, RPT)])


def _scores(src, dst, ssrc, sdst):
    f = pl.kernel(
        _scores_body,
        out_type=[
            jax.ShapeDtypeStruct((E, 16), _f32),
            jax.ShapeDtypeStruct((NC, N, 16), _f32),
        ],
        mesh=_mesh(),
        scratch_types=[
            pltpu.VMEM((K,), _i32),
            pltpu.VMEM((K,), _i32),
            pltpu.VMEM((K, 16), _f32),
            pltpu.VMEM((K, 16), _f32),
            pltpu.VMEM((K, 16), _f32),
            pltpu.VMEM((RPT, 16), _f32),
            pltpu.VMEM_SHARED((N, 16), _f32),
            pltpu.SemaphoreType.DMA,
            pltpu.SemaphoreType.DMA,
        ],
    )
    return f(src, dst, ssrc, sdst)


def _recip_body(p0_ref, p1_ref, r_ref):
    r_ref[...] = 1.0 / (p0_ref[0] + p1_ref[0] + 1e-16)


def _recip(esum, blk=400):
    grid = (N // blk,)
    return pl.pallas_call(
        _recip_body,
        grid=grid,
        in_specs=[
            pl.BlockSpec((1, blk, 16), lambda i: (0, i, 0)),
            pl.BlockSpec((1, blk, 16), lambda i: (1, i, 0)),
        ],
        out_specs=pl.BlockSpec((blk, 16), lambda i: (i, 0)),
        out_shape=jax.ShapeDtypeStruct((N, 16), _f32),
    )(esum, esum)


def _agg_body(src_hbm, dst_hbm, wh_hbm, r_hbm, eexp_hbm,
              outp_hbm,
              sidx, didx, whbuf, rbuf, ebuf, obuf, zbuf, out_sh, sem1, sem2):
    c = lax.axis_index("c")
    s = lax.axis_index("s")

    def zrow(i, carry):
        for j in range(HD // 16):
            zbuf[i, pl.ds(16 * j, 16)] = jnp.zeros((16,), _f32)
        return carry
    lax.fori_loop(0, 125, zrow, 0)
    for j in range(RPT // 125):
        pltpu.sync_copy(zbuf, out_sh.at[pl.ds(s * RPT + j * 125, 125)])
    plsc.subcore_barrier()

    base0 = (c * NS + s) * EPW

    def chunk(ci, carry):
        base = base0 + ci * K
        pltpu.sync_copy(src_hbm.at[pl.ds(base, K)], sidx)
        pltpu.sync_copy(dst_hbm.at[pl.ds(base, K)], didx)
        h1 = pltpu.async_copy(wh_hbm.at[sidx], whbuf, sem1)
        h2 = pltpu.async_copy(r_hbm.at[didx], rbuf, sem2)
        pltpu.sync_copy(eexp_hbm.at[pl.ds(base, K)], ebuf)
        h1.wait()
        h2.wait()

        def edge(k, carry2):
            fv = ebuf[k, :] * rbuf[k, :]
            for j in range(HD // 16):
                obuf[k, pl.ds(16 * j, 16)] = whbuf[k, pl.ds(16 * j, 16)] * fv
            return carry2
        lax.fori_loop(0, K, edge, 0)

        pltpu.sync_copy(obuf, out_sh.at[didx], add=True)
        return carry
    lax.fori_loop(0, NCHUNK, chunk, 0)

    plsc.subcore_barrier()
    pltpu.sync_copy(out_sh.at[pl.ds(s * RPT, RPT)],
                    outp_hbm.at[c, pl.ds(s * RPT, RPT)])


def _aggregate(src, dst, wh, r, eexp):
    f = pl.kernel(
        _agg_body,
        out_type=jax.ShapeDtypeStruct((NC, N, HD), _f32),
        mesh=_mesh(),
        scratch_types=[
            pltpu.VMEM((K,), _i32),
            pltpu.VMEM((K,), _i32),
            pltpu.VMEM((K, HD), _f32),
            pltpu.VMEM((K, 16), _f32),
            pltpu.VMEM((K, 16), _f32),
            pltpu.VMEM((K, HD), _f32),
            pltpu.VMEM((125, HD), _f32),
            pltpu.VMEM_SHARED((N, HD), _f32),
            pltpu.SemaphoreType.DMA,
            pltpu.SemaphoreType.DMA,
        ],
    )
    return f(src, dst, wh, r, eexp)


def _final_body(o0_ref, o1_ref, p_ref, b_ref, out_ref):
    y = o0_ref[0] + o1_ref[0]
    out_ref[...] = (
        jnp.dot(y, p_ref[...], preferred_element_type=_f32) + b_ref[...]
    )


def _finalize(outp, perm_mat, bias, blk=400):
    grid = (N // blk,)
    return pl.pallas_call(
        _final_body,
        grid=grid,
        in_specs=[
            pl.BlockSpec((1, blk, HD), lambda i: (0, i, 0)),
            pl.BlockSpec((1, blk, HD), lambda i: (1, i, 0)),
            pl.BlockSpec((HD, HD), lambda i: (0, 0)),
            pl.BlockSpec((1, HD), lambda i: (0, 0)),
        ],
        out_specs=pl.BlockSpec((blk, HD), lambda i: (i, 0)),
        out_shape=jax.ShapeDtypeStruct((N, HD), _f32),
    )(outp, outp, perm_mat, bias)


_j = np.arange(HD)
_DM_FROM_STD = (_j % H) * D + _j // H
_STD_FROM_DM = (_j % D) * H + _j // D
_PERM = np.zeros((HD, HD), dtype=np.float32)
_PERM[_STD_FROM_DM, _j] = 1.0


@jax.jit
def kernel(x, src, dst, W, a, bias):
    wp = W[:, _DM_FROM_STD]
    a1 = a[:, :D]
    a2 = a[:, D:]
    eye = np.equal.outer(np.arange(H), np.arange(H)).astype(np.float32)
    msrc = (a1[:, :, None] * eye[:, None, :]).reshape(HD, H)
    mdst = (a2[:, :, None] * eye[:, None, :]).reshape(HD, H)
    wm = jnp.concatenate(
        [W @ msrc, W @ msrc, W @ mdst, W @ mdst], axis=1)

    wh_dm, ssrc, sdst = _precompute(x, wp, wm)
    eexp, esum = _scores(src, dst, ssrc, sdst)
    r = _recip(esum)
    outp = _aggregate(src, dst, wh_dm, r, eexp)
    return _finalize(outp, jnp.asarray(_PERM), bias.reshape(1, HD))


# speedup vs baseline: 80.4448x; 80.4448x over previous
import jax
import jax.numpy as jnp
import numpy as np
from jax import lax
from jax.experimental import pallas as pl
from jax.experimental.pallas import tpu as pltpu
from jax.experimental.pallas import tpu_sc as plsc

N = 10000
E = 320000
IN_DIM = 128
H = 8
D = 16
HD = H * D
LEAKY_SLOPE = 0.2

NC = 2
NS = 16
NW = NC * NS
EPW = E // NW
K = 80
NCHUNK = EPW // K
N_PAD = 10240
RPT = N_PAD // NS
ZB = 128

_f32 = jnp.float32
_i32 = jnp.int32


def _mm_body(x_ref, wp_ref, wm_ref, wh_ref, ssrc_ref, sdst_ref):
    xb = x_ref[...]
    wh_ref[...] = jnp.dot(xb, wp_ref[...], preferred_element_type=_f32)
    s = jnp.dot(xb, wm_ref[...], preferred_element_type=_f32)
    ssrc_ref[...] = s[:, :16]
    sdst_ref[...] = s[:, 16:]


def _precompute(x_pad, wp, wm, blk=512):
    grid = (N_PAD // blk,)
    return pl.pallas_call(
        _mm_body,
        grid=grid,
        in_specs=[
            pl.BlockSpec((blk, IN_DIM), lambda i: (i, 0)),
            pl.BlockSpec((IN_DIM, HD), lambda i: (0, 0)),
            pl.BlockSpec((IN_DIM, 32), lambda i: (0, 0)),
        ],
        out_specs=[
            pl.BlockSpec((blk, HD), lambda i: (i, 0)),
            pl.BlockSpec((blk, 16), lambda i: (i, 0)),
            pl.BlockSpec((blk, 16), lambda i: (i, 0)),
        ],
        out_shape=[
            jax.ShapeDtypeStruct((N_PAD, HD), _f32),
            jax.ShapeDtypeStruct((N_PAD, 16), _f32),
            jax.ShapeDtypeStruct((N_PAD, 16), _f32),
        ],
    )(x_pad, wp, wm)


def _edge_body(src_hbm, dst_hbm, ssrc_hbm, sdst_hbm, wh_hbm,
               esum_hbm, u_hbm,
               sidx, didx, sbuf, dbuf, ebuf, whbuf, obuf,
               esum_sh, u_sh,
               sem1, sem2, sem3):
    c = lax.axis_index("c")
    s = lax.axis_index("s")
    rows = pl.ds(s * RPT, RPT)

    def zrow(i, carry):
        ebuf[i, :] = jnp.zeros((16,), _f32)
        for j in range(HD // 16):
            obuf[i, pl.ds(16 * j, 16)] = jnp.zeros((16,), _f32)
        return carry
    lax.fori_loop(0, K, zrow, 0)
    for j in range(RPT // K):
        pltpu.sync_copy(obuf, u_sh.at[pl.ds(s * RPT + j * K, K)])
        pltpu.sync_copy(ebuf, esum_sh.at[pl.ds(s * RPT + j * K, K)])
    plsc.subcore_barrier()

    base0 = (c * NS + s) * EPW

    def chunk(ci, carry):
        base = base0 + ci * K
        pltpu.sync_copy(src_hbm.at[pl.ds(base, K)], sidx)
        pltpu.sync_copy(dst_hbm.at[pl.ds(base, K)], didx)
        h1 = pltpu.async_copy(wh_hbm.at[sidx], whbuf, sem1)
        h2 = pltpu.async_copy(ssrc_hbm.at[sidx], sbuf, sem2)
        h3 = pltpu.async_copy(sdst_hbm.at[didx], dbuf, sem3)
        h2.wait()
        h3.wait()

        def escore(k, carry2):
            e = sbuf[k, :] + dbuf[k, :]
            e = jnp.where(e >= 0.0, e, LEAKY_SLOPE * e)
            ebuf[k, :] = jnp.exp(e)
            return carry2
        lax.fori_loop(0, K, escore, 0)
        h1.wait()

        def weight(k, carry2):
            fv = ebuf[k, :]
            for j in range(HD // 16):
                obuf[k, pl.ds(16 * j, 16)] = whbuf[k, pl.ds(16 * j, 16)] * fv
            return carry2
        lax.fori_loop(0, K, weight, 0)

        pltpu.sync_copy(ebuf, esum_sh.at[didx], add=True)
        pltpu.sync_copy(obuf, u_sh.at[didx], add=True)
        return carry
    lax.fori_loop(0, NCHUNK, chunk, 0)

    plsc.subcore_barrier()
    pltpu.sync_copy(esum_sh.at[rows], esum_hbm.at[c, rows])
    pltpu.sync_copy(u_sh.at[rows], u_hbm.at[c, rows])


def _edge_phase(src, dst, ssrc, sdst, wh):
    f = pl.kernel(
        _edge_body,
        out_type=[
            jax.ShapeDtypeStruct((NC, N_PAD, 16), _f32),
            jax.ShapeDtypeStruct((NC, N_PAD, HD), _f32),
        ],
        mesh=plsc.VectorSubcoreMesh(core_axis_name="c", subcore_axis_name="s"),
        compiler_params=pltpu.CompilerParams(use_tc_tiling_on_sc=False),
        scratch_types=[
            pltpu.VMEM((K,), _i32),
            pltpu.VMEM((K,), _i32),
            pltpu.VMEM((K, 16), _f32),
            pltpu.VMEM((K, 16), _f32),
            pltpu.VMEM((K, 16), _f32),
            pltpu.VMEM((K, HD), _f32),
            pltpu.VMEM((K, HD), _f32),
            pltpu.VMEM_SHARED((N_PAD, 16), _f32),
            pltpu.VMEM_SHARED((N_PAD, HD), _f32),
            pltpu.SemaphoreType.DMA,
            pltpu.SemaphoreType.DMA,
            pltpu.SemaphoreType.DMA,
        ],
    )
    return f(src, dst, ssrc, sdst, wh)


def _final_body(e0_ref, e1_ref, u0_ref, u1_ref, p_ref, b_ref, out_ref):
    r = 1.0 / (e0_ref[0] + e1_ref[0] + 1e-16)
    rr = jnp.concatenate([r] * (HD // 16), axis=1)
    y = (u0_ref[0] + u1_ref[0]) * rr
    out_ref[...] = (
        jnp.dot(y, p_ref[...], preferred_element_type=_f32) + b_ref[...]
    )


def _finalize(esum, u, perm_mat, bias, blk=400):
    grid = (N // blk,)
    return pl.pallas_call(
        _final_body,
        grid=grid,
        in_specs=[
            pl.BlockSpec((1, blk, 16), lambda i: (0, i, 0)),
            pl.BlockSpec((1, blk, 16), lambda i: (1, i, 0)),
            pl.BlockSpec((1, blk, HD), lambda i: (0, i, 0)),
            pl.BlockSpec((1, blk, HD), lambda i: (1, i, 0)),
            pl.BlockSpec((HD, HD), lambda i: (0, 0)),
            pl.BlockSpec((1, HD), lambda i: (0, 0)),
        ],
        out_specs=pl.BlockSpec((blk, HD), lambda i: (i, 0)),
        out_shape=jax.ShapeDtypeStruct((N, HD), _f32),
    )(esum, esum, u, u, perm_mat, bias)


_j = np.arange(HD)
_DM_FROM_STD = (_j % H) * D + _j // H
_STD_FROM_DM = (_j % D) * H + _j // D
_PERM = np.zeros((HD, HD), dtype=np.float32)
_PERM[_STD_FROM_DM, _j] = 1.0


@jax.jit
def kernel(x, src, dst, W, a, bias):
    wp = W[:, _DM_FROM_STD]
    a1 = a[:, :D]
    a2 = a[:, D:]
    eye = np.equal.outer(np.arange(H), np.arange(H)).astype(np.float32)
    msrc = (a1[:, :, None] * eye[:, None, :]).reshape(HD, H)
    mdst = (a2[:, :, None] * eye[:, None, :]).reshape(HD, H)
    wm = jnp.concatenate(
        [W @ msrc, W @ msrc, W @ mdst, W @ mdst], axis=1)

    x_pad = jnp.pad(x, ((0, N_PAD - N), (0, 0)))
    wh_dm, ssrc, sdst = _precompute(x_pad, wp, wm)
    esum, u = _edge_phase(src, dst, ssrc, sdst, wh_dm)
    return _finalize(esum, u, jnp.asarray(_PERM), bias.reshape(1, HD))
